# baseline (speedup 1.0000x reference)
import jax
import jax.numpy as jnp
from jax.experimental import pallas as pl


def _mlp2(v, w1, b1, w2, b2):
    h = jax.nn.relu(v @ w1 + b1)
    return h @ w2 + b2


def _gcn_conv(x, row, col, N, W, b):
    sl = jnp.arange(N)
    r = jnp.concatenate([row, sl])
    c = jnp.concatenate([col, sl])
    deg = jax.ops.segment_sum(jnp.ones(r.shape[0], x.dtype), r, num_segments=N)
    dinv = jnp.where(deg > 0, 1.0 / jnp.sqrt(deg), 0.0)
    norm = dinv[r] * dinv[c]
    xw = x @ W
    out = jax.ops.segment_sum(xw[c] * norm[:, None], r, num_segments=N)
    return out + b


def _combine_kernel(s0_ref, gp_w1_ref, gp_b1_ref, gp_w2_ref, gp_b2_ref,
                    feat_ref, al_ref, out_ref, os_ref, raw_ref):
    s0 = s0_ref[...]
    h = jax.nn.relu(s0 * gp_w1_ref[...] + gp_b1_ref[...])
    raw = jnp.sum(h * gp_w2_ref[...], axis=1, keepdims=True) + gp_b2_ref[0, 0]
    os_ = jax.nn.sigmoid(raw)
    al = al_ref[...]
    raw_ref[...] = raw
    os_ref[...] = os_
    out_ref[...] = al[0, 0] * os_ + al[0, 1] * feat_ref[...] + 1e-15


def kernel(edge, edge_index, edge_weight, x, W1, b1, W2, b2, W3, b3,
           fe_w1, fe_b1, fe_w2, fe_b2, fn_w1, fn_b1, fn_w2, fn_b2,
           gp_w1, gp_b1, gp_w2, gp_b2, alpha, num_nodes):
    N = x.shape[0]
    B = edge.shape[1]
    row, col = edge_index[0], edge_index[1]
    h = jax.nn.relu(_gcn_conv(x, row, col, N, W1, b1))
    h = jax.nn.relu(_gcn_conv(h, row, col, N, W2, b2))
    h = _gcn_conv(h, row, col, N, W3, b3)
    out_feat = jnp.sum(h[edge[0]] * h[edge[1]], axis=0)

    ew = _mlp2(edge_weight[:, None], fe_w1, fe_b1, fe_w2, fe_b2)
    nsf = jax.ops.segment_sum(ew, col, num_segments=N)
    wprime = edge_weight * jnp.squeeze(
        _mlp2(nsf, fn_w1, fn_b1, fn_w2, fn_b2), -1)[col]
    A_dense = jnp.zeros((N, N), edge_weight.dtype).at[row, col].add(wprime)
    out_struct0 = jnp.sum(A_dense[edge[0]] * A_dense[edge[1]], axis=1)

    al = jax.nn.softmax(alpha)
    gp = gp_w1.shape[1]
    out, out_struct, out_struct_raw = pl.pallas_call(
        _combine_kernel,
        out_shape=(
            jax.ShapeDtypeStruct((B, 256), jnp.float32),
            jax.ShapeDtypeStruct((B, 1), jnp.float32),
            jax.ShapeDtypeStruct((B, 1), jnp.float32),
        ),
    )(out_struct0[:, None], gp_w1, gp_b1[None, :], gp_w2.T, gp_b2[None, :],
      out_feat[None, :], al[None, :])
    return (out, out_struct, out_feat, out_struct_raw)


# speedup vs baseline: 4.8612x; 4.8612x over previous
import functools

import jax
import jax.numpy as jnp
from jax import lax
from jax.experimental import pallas as pl
from jax.experimental.pallas import tpu as pltpu, tpu_sc as plsc

N = 10000
NB = 10240
E = 160000
EP = 163840
B = 2048
NW = 32
EPW = EP // NW
EPT = EP // 16

_MESH = lambda: plsc.VectorSubcoreMesh(core_axis_name="c", subcore_axis_name="s")
_CP = lambda: pltpu.CompilerParams(needs_layout_passes=False,
                                   use_tc_tiling_on_sc=False)


def _z16():
    return jnp.zeros((16,), jnp.float32)


def _iota16():
    return lax.iota(jnp.int32, 16)


def _zero2d(ref, rows):
    z = jnp.zeros((16,), ref.dtype)
    for i in range(rows):
        for j in range(8):
            ref[i, j * 16:(j + 1) * 16] = z


def _count_kernel(row_hbm, col_hbm, ew_hbm, cnt_hbm, nsf_hbm,
                  rowc, colc, wc, cnth, nsfh):
    cid = lax.axis_index("c")
    sid = lax.axis_index("s")
    w = sid * 2 + cid
    _zero2d(cnth, 80)
    _zero2d(nsfh, 80)

    @pl.loop(0, 40)
    def _chunk(g):
        eb = w * EPW + g * 128
        pltpu.sync_copy(row_hbm.at[pl.ds(eb, 128)], rowc)
        pltpu.sync_copy(col_hbm.at[pl.ds(eb, 128)], colc)
        pltpu.sync_copy(ew_hbm.at[pl.ds(eb, 128)], wc)
        for sg in range(8):
            sl = pl.ds(sg * 16, 16)
            rows = rowc[sl]
            cols = colc[sl]
            ew = wc[sl]
            hi, lo = rows >> 7, rows & 127
            cnt, last = plsc.scan_count(rows)
            cur = plsc.load_gather(cnth, [hi, lo])
            plsc.store_scatter(cnth, [hi, lo], cur + cnt, mask=last)
            chi, clo = cols >> 7, cols & 127
            cnt2, _ = plsc.scan_count(cols)
            for k in range(16):
                plsc.addupdate_scatter(nsfh, [chi, clo], ew,
                                       mask=(cnt2 == (k + 1)))

    pltpu.sync_copy(cnth, cnt_hbm.at[w])
    pltpu.sync_copy(nsfh, nsf_hbm.at[w])


def _run_count(row1d, col1d, ewmlp1d):
    kern = functools.partial(
        pl.kernel, _count_kernel,
        out_type=(jax.ShapeDtypeStruct((NW, 80, 128), jnp.int32),
                  jax.ShapeDtypeStruct((NW, 80, 128), jnp.float32)),
        mesh=_MESH(), compiler_params=_CP(),
        scratch_types=[pltpu.VMEM((128,), jnp.int32),
                       pltpu.VMEM((128,), jnp.int32),
                       pltpu.VMEM((128,), jnp.float32),
                       pltpu.VMEM((80, 128), jnp.int32),
                       pltpu.VMEM((80, 128), jnp.float32)],
    )()
    return kern(row1d, col1d, ewmlp1d)


def _scan_kernel(cnt_ref, nsf_ref, dinv_ref, bases_ref, base_ref, tot_ref,
                 nsfall_ref):
    cacc = cnt_ref[0]
    nacc = nsf_ref[0]
    for t in range(1, NW):
        cacc = cacc + cnt_ref[t]
        nacc = nacc + nsf_ref[t]
    incl = cacc
    for sh in (1, 2, 4, 8, 16, 32, 64):
        z = jnp.zeros((80, sh), jnp.int32)
        incl = incl + jnp.concatenate([z, incl[:, :128 - sh]], axis=1)
    rowtot = jnp.sum(cacc, axis=1, keepdims=True)
    rowincl = rowtot
    for sh in (1, 2, 4, 8, 16, 32, 64):
        z = jnp.zeros((sh, 1), jnp.int32)
        rowincl = rowincl + jnp.concatenate([z, rowincl[:80 - sh, :]], axis=0)
    rowpre = rowincl - rowtot
    base = rowpre + incl - cacc
    running = jnp.zeros((80, 128), jnp.int32)
    for t in range(NW):
        bases_ref[t] = base + running
        running = running + cnt_ref[t]
    dinv_ref[...] = lax.rsqrt(cacc.astype(jnp.float32) + 1.0)
    base_ref[...] = base
    tot_ref[...] = cacc
    nsfall_ref[...] = nacc


def _run_scan(cnt_t, nsf_t):
    return pl.pallas_call(
        _scan_kernel,
        out_shape=(jax.ShapeDtypeStruct((80, 128), jnp.float32),
                   jax.ShapeDtypeStruct((NW, 80, 128), jnp.int32),
                   jax.ShapeDtypeStruct((80, 128), jnp.int32),
                   jax.ShapeDtypeStruct((80, 128), jnp.int32),
                   jax.ShapeDtypeStruct((80, 128), jnp.float32)),
    )(cnt_t, nsf_t)


def _rowmlp_kernel(v_ref, w1_ref, b1_ref, w2_ref, b2_ref, out_ref):
    h = jax.nn.relu(v_ref[...] * w1_ref[...] + b1_ref[...])
    out_ref[...] = jnp.sum(h * w2_ref[...], axis=1, keepdims=True) + b2_ref[0, 0]


def _run_rowmlp(vcol, w1, b1, w2, b2):
    n = vcol.shape[0]
    return pl.pallas_call(
        _rowmlp_kernel,
        out_shape=jax.ShapeDtypeStruct((n, 1), jnp.float32),
    )(vcol, w1.reshape(1, -1), b1.reshape(1, -1), w2.reshape(1, -1),
      b2.reshape(1, 1))


def _mm1_kernel(x_ref, w_ref, dinv_ref, y_ref):
    y_ref[...] = jnp.dot(x_ref[...], w_ref[...],
                         preferred_element_type=jnp.float32) * dinv_ref[...]


def _run_mm1(x_pad, W1, dinvb):
    return pl.pallas_call(
        _mm1_kernel,
        grid=(10, 2),
        in_specs=[pl.BlockSpec((1024, 128), lambda i, h: (i, 0)),
                  pl.BlockSpec((128, 128), lambda i, h: (0, h)),
                  pl.BlockSpec((1024, 128), lambda i, h: (i, 0))],
        out_specs=pl.BlockSpec((1024, 128), lambda i, h: (h * 10 + i, 0)),
        out_shape=jax.ShapeDtypeStruct((2 * NB, 128), jnp.float32),
    )(x_pad, W1, dinvb)


def _comb_mm_kernel(a0_ref, a1_ref, y0_ref, y1_ref, dinv_ref, b_ref, w_ref,
                    yo_ref):
    d = dinv_ref[...]
    g0 = jax.nn.relu(d * (a0_ref[...] + y0_ref[...]) + b_ref[:, 0:128])
    g1 = jax.nn.relu(d * (a1_ref[...] + y1_ref[...]) + b_ref[:, 128:256])
    yo_ref[...] = (jnp.dot(g0, w_ref[0:128, :],
                           preferred_element_type=jnp.float32)
                   + jnp.dot(g1, w_ref[128:256, :],
                             preferred_element_type=jnp.float32)) * d


def _run_comb_mm(acc, y, dinvb, b, W):
    lo = pl.BlockSpec((1024, 128), lambda i, h: (i, 0))
    hi = pl.BlockSpec((1024, 128), lambda i, h: (10 + i, 0))
    return pl.pallas_call(
        _comb_mm_kernel,
        grid=(10, 2),
        in_specs=[lo, hi, lo, hi,
                  pl.BlockSpec((1024, 128), lambda i, h: (i, 0)),
                  pl.BlockSpec((1, 256), lambda i, h: (0, 0)),
                  pl.BlockSpec((256, 128), lambda i, h: (0, h))],
        out_specs=pl.BlockSpec((1024, 128), lambda i, h: (h * 10 + i, 0)),
        out_shape=jax.ShapeDtypeStruct((2 * NB, 128), jnp.float32),
    )(acc, acc, y, y, dinvb, b.reshape(1, 256), W)


def _comb3_kernel(a_ref, y_ref, dinv_ref, b_ref, h_ref):
    h_ref[...] = dinv_ref[...] * (a_ref[...] + y_ref[...]) + b_ref[...]


def _run_comb3(acc, y, dinvb, b3):
    half = pl.BlockSpec((1024, 128), lambda i, h: (h * 10 + i, 0))
    return pl.pallas_call(
        _comb3_kernel,
        grid=(10, 2),
        in_specs=[half, half,
                  pl.BlockSpec((1024, 128), lambda i, h: (i, 0)),
                  pl.BlockSpec((1, 128), lambda i, h: (0, h))],
        out_specs=half,
        out_shape=jax.ShapeDtypeStruct((2 * NB, 128), jnp.float32),
    )(acc, y, dinvb, b3.reshape(1, 256))


def _conv_kernel(y_hbm, row_hbm, col_hbm, acc_hbm, rowc, colc, dbuf, zbuf,
                 acc_sh, sem):
    cid = lax.axis_index("c")
    sid = lax.axis_index("s")
    _zero2d(zbuf, 128)
    for k in range(5):
        pltpu.sync_copy(zbuf, acc_sh.at[pl.ds(sid * 640 + k * 128, 128)])
    off = cid * NB
    plsc.subcore_barrier()

    @pl.loop(0, 80)
    def _chunk(g):
        eb = sid * EPT + g * 128
        pltpu.sync_copy(row_hbm.at[pl.ds(eb, 128)], rowc)
        pltpu.sync_copy(col_hbm.at[pl.ds(eb, 128)], colc)
        for j in range(8):
            sl = pl.ds(j * 16, 16)
            colc[sl] = colc[sl] + off
        pltpu.async_copy(y_hbm.at[colc], dbuf, sem).wait()
        pltpu.sync_copy(dbuf, acc_sh.at[rowc], add=True)

    plsc.subcore_barrier()
    pltpu.sync_copy(acc_sh.at[pl.ds(sid * 640, 640)],
                    acc_hbm.at[pl.ds(off + sid * 640, 640)])


def _run_conv(y_flat, row1d, col1d):
    kern = functools.partial(
        pl.kernel, _conv_kernel,
        out_type=jax.ShapeDtypeStruct((2 * NB, 128), jnp.float32),
        mesh=_MESH(), compiler_params=_CP(),
        scratch_types=[pltpu.VMEM((128,), jnp.int32),
                       pltpu.VMEM((128,), jnp.int32),
                       pltpu.VMEM((128, 128), jnp.float32),
                       pltpu.VMEM((128, 128), jnp.float32),
                       pltpu.VMEM_SHARED((NB, 128), jnp.float32),
                       pltpu.SemaphoreType.DMA],
    )()
    return kern(y_flat, row1d, col1d)


def _place_kernel(row_hbm, col_hbm, ew_hbm, fn_hbm, bases_hbm, rec_hbm,
                  rowc, colc, wc, fnb, cur, stg, posv, sem):
    cid = lax.axis_index("c")
    sid = lax.axis_index("s")
    w = sid * 2 + cid
    pltpu.sync_copy(fn_hbm, fnb)
    pltpu.sync_copy(bases_hbm.at[w], cur)
    it16 = _iota16()

    @pl.loop(0, 40)
    def _chunk(g):
        eb = w * EPW + g * 128
        pltpu.sync_copy(row_hbm.at[pl.ds(eb, 128)], rowc)
        pltpu.sync_copy(col_hbm.at[pl.ds(eb, 128)], colc)
        pltpu.sync_copy(ew_hbm.at[pl.ds(eb, 128)], wc)
        for sg in range(8):
            sl = pl.ds(sg * 16, 16)
            rows = rowc[sl]
            cols = colc[sl]
            wv = wc[sl]
            wp = wv * plsc.load_gather(fnb, [cols >> 7, cols & 127])
            hi, lo = rows >> 7, rows & 127
            cnt, last = plsc.scan_count(rows)
            curv = plsc.load_gather(cur, [hi, lo])
            pos = curv + cnt - 1
            plsc.store_scatter(cur, [hi, lo], curv + cnt, mask=last)
            ridx = it16 + sg * 16
            plsc.store_scatter(stg, [ridx, jnp.zeros((16,), jnp.int32)],
                               plsc.bitcast(cols, jnp.float32))
            plsc.store_scatter(stg, [ridx, jnp.ones((16,), jnp.int32)], wp)
            plsc.store_scatter(posv, [ridx], pos)
        pltpu.async_copy(stg, rec_hbm.at[posv], sem).wait()


def _run_place(row1d, col1d, ew1d, fnode2d, bases_t):
    kern = functools.partial(
        pl.kernel, _place_kernel,
        out_type=jax.ShapeDtypeStruct((EP, 16), jnp.float32),
        mesh=_MESH(), compiler_params=_CP(),
        scratch_types=[pltpu.VMEM((128,), jnp.int32),
                       pltpu.VMEM((128,), jnp.int32),
                       pltpu.VMEM((128,), jnp.float32),
                       pltpu.VMEM((80, 128), jnp.float32),
                       pltpu.VMEM((80, 128), jnp.int32),
                       pltpu.VMEM((128, 16), jnp.float32),
                       pltpu.VMEM((128,), jnp.int32),
                       pltpu.SemaphoreType.DMA],
    )()
    return kern(row1d, col1d, ew1d, fnode2d, bases_t)


def _isect_kernel(src_hbm, dst_hbm, base_hbm, tot_hbm, rec_hbm, out_hbm,
                  srcb, dstb, baseb, totb, dense, recb, idxv, resm, sem):
    cid = lax.axis_index("c")
    sid = lax.axis_index("s")
    w = sid * 2 + cid
    pltpu.sync_copy(src_hbm, srcb.at[pl.ds(0, B)])
    pltpu.sync_copy(dst_hbm, dstb.at[pl.ds(0, B)])
    pltpu.sync_copy(base_hbm, baseb)
    pltpu.sync_copy(tot_hbm, totb)
    _zero2d(dense, 80)
    it16 = _iota16()
    zer = _z16()

    def cand(i, _):
        s_id = srcb[pl.ds(w * 64 + i, 16)][0]
        d_id = dstb[pl.ds(w * 64 + i, 16)][0]
        bs = baseb[pl.ds(s_id, 16)][0]
        cs = totb[pl.ds(s_id, 16)][0]
        bd = baseb[pl.ds(d_id, 16)][0]
        cd = totb[pl.ds(d_id, 16)][0]

        def read_chunk(basep, cnp, ci):
            raw = basep + ci * 16 + it16
            lim = basep + cnp - 1
            idxv[...] = jnp.minimum(raw, lim)
            pltpu.async_copy(rec_hbm.at[idxv], recb, sem).wait()
            lm = (ci * 16 + it16) < cnp
            colsf = plsc.load_gather(recb, [it16, jnp.zeros((16,), jnp.int32)])
            cols = plsc.bitcast(colsf, jnp.int32)
            wv = plsc.load_gather(recb, [it16, jnp.ones((16,), jnp.int32)])
            colskey = jnp.where(lm, cols, NB + it16)
            ghi = jnp.where(lm, cols >> 7, 0)
            glo = jnp.where(lm, cols & 127, 0)
            return lm, colskey, ghi, glo, wv

        ncd = (cd + 15) >> 4

        def dbody(ci, _c):
            lm, colskey, ghi, glo, wv = read_chunk(bd, cd, ci)
            cnt2, _l = plsc.scan_count(colskey)
            wm = jnp.where(lm, wv, 0.0)
            for k in range(16):
                plsc.addupdate_scatter(dense, [ghi, glo], wm,
                                       mask=jnp.logical_and(lm, cnt2 == (k + 1)))
            return 0

        lax.fori_loop(0, ncd, dbody, 0)

        ncs = (cs + 15) >> 4

        def sbody(ci, acc):
            lm, _ck, ghi, glo, wv = read_chunk(bs, cs, ci)
            vals = plsc.load_gather(dense, [ghi, glo])
            return acc + jnp.where(lm, wv * vals, 0.0)

        part = lax.fori_loop(0, ncs, sbody, zer)

        def cbody(ci, _c):
            lm, colskey, ghi, glo, _w = read_chunk(bd, cd, ci)
            _c2, lastx = plsc.scan_count(colskey)
            plsc.store_scatter(dense, [ghi, glo], zer,
                               mask=jnp.logical_and(lm, lastx))
            return 0

        lax.fori_loop(0, ncd, cbody, 0)
        plsc.store_scatter(resm, [jnp.full((16,), i, jnp.int32), it16], part)
        return 0

    lax.fori_loop(0, 64, cand, 0)
    pltpu.sync_copy(resm, out_hbm.at[pl.ds(w * 64, 64)])


def _run_isect(src1d, dst1d, base1d, tot1d, records):
    kern = functools.partial(
        pl.kernel, _isect_kernel,
        out_type=jax.ShapeDtypeStruct((B, 16), jnp.float32),
        mesh=_MESH(), compiler_params=_CP(),
        scratch_types=[pltpu.VMEM((B + 16,), jnp.int32),
                       pltpu.VMEM((B + 16,), jnp.int32),
                       pltpu.VMEM((NB,), jnp.int32),
                       pltpu.VMEM((NB,), jnp.int32),
                       pltpu.VMEM((80, 128), jnp.float32),
                       pltpu.VMEM((16, 16), jnp.float32),
                       pltpu.VMEM((16,), jnp.int32),
                       pltpu.VMEM((64, 16), jnp.float32),
                       pltpu.SemaphoreType.DMA],
    )()
    return kern(src1d, dst1d, base1d, tot1d, records)


def _outfeat_kernel(h_hbm, src_hbm, dst_hbm, out_hbm,
                    srcb, dstb, sidx, didx, sbuf, dbuf, resv, sem, sem2):
    cid = lax.axis_index("c")
    sid = lax.axis_index("s")
    pltpu.sync_copy(src_hbm, srcb)
    pltpu.sync_copy(dst_hbm, dstb)
    off = cid * NB
    accs = [_z16() for _ in range(8)]
    for ch in range(8):
        sl = pl.ds(sid * 128 + ch * 16, 16)
        sidx[...] = srcb[sl] + off
        didx[...] = dstb[sl] + off
        pltpu.async_copy(h_hbm.at[sidx], sbuf, sem).wait()
        pltpu.async_copy(h_hbm.at[didx], dbuf, sem2).wait()
        for p in range(16):
            for seg in range(8):
                ssl = pl.ds(seg * 16, 16)
                accs[seg] = accs[seg] + sbuf[p, ssl] * dbuf[p, ssl]
    for seg in range(8):
        resv[pl.ds(seg * 16, 16)] = accs[seg]
    pltpu.sync_copy(resv, out_hbm.at[cid * 16 + sid])


def _run_outfeat(h3_flat, src1d, dst1d):
    kern = functools.partial(
        pl.kernel, _outfeat_kernel,
        out_type=jax.ShapeDtypeStruct((32, 128), jnp.float32),
        mesh=_MESH(), compiler_params=_CP(),
        scratch_types=[pltpu.VMEM((B,), jnp.int32),
                       pltpu.VMEM((B,), jnp.int32),
                       pltpu.VMEM((16,), jnp.int32),
                       pltpu.VMEM((16,), jnp.int32),
                       pltpu.VMEM((16, 128), jnp.float32),
                       pltpu.VMEM((16, 128), jnp.float32),
                       pltpu.VMEM((128,), jnp.float32),
                       pltpu.SemaphoreType.DMA,
                       pltpu.SemaphoreType.DMA],
    )()
    return kern(h3_flat, src1d, dst1d)


def _final_kernel(s0_ref, part_ref, s0o_ref, feat_ref):
    s0o_ref[...] = jnp.sum(s0_ref[...], axis=1, keepdims=True)
    p = part_ref[...]
    f0 = jnp.sum(p[0:16], axis=0, keepdims=True)
    f1 = jnp.sum(p[16:32], axis=0, keepdims=True)
    feat_ref[...] = jnp.concatenate([f0, f1], axis=1)


def _run_final(out0, partials):
    return pl.pallas_call(
        _final_kernel,
        out_shape=(jax.ShapeDtypeStruct((B, 1), jnp.float32),
                   jax.ShapeDtypeStruct((1, 256), jnp.float32)),
    )(out0, partials)


def _mlp2(v, w1, b1, w2, b2):
    h = jax.nn.relu(v @ w1 + b1)
    return h @ w2 + b2


def kernel(edge, edge_index, edge_weight, x, W1, b1, W2, b2, W3, b3,
           fe_w1, fe_b1, fe_w2, fe_b2, fn_w1, fn_b1, fn_w2, fn_b2,
           gp_w1, gp_b1, gp_w2, gp_b2, alpha, num_nodes):
    row = edge_index[0].astype(jnp.int32)
    col = edge_index[1].astype(jnp.int32)
    npad = EP - E
    row1d = jnp.concatenate([row, jnp.full((npad,), 10016, jnp.int32)])
    col1d = jnp.concatenate([col, jnp.zeros((npad,), jnp.int32)])
    ew1d = jnp.concatenate([edge_weight, jnp.zeros((npad,), jnp.float32)])

    ewm = _mlp2(edge_weight[:, None], fe_w1, fe_b1, fe_w2, fe_b2)[:, 0]
    ewm1d = jnp.concatenate([ewm, jnp.zeros((npad,), jnp.float32)])

    cnt_t, nsf_t = _run_count(row1d, col1d, ewm1d)
    dinv2d, bases_t, base2d, tot2d, nsf2d = _run_scan(cnt_t, nsf_t)

    dinvb = jnp.broadcast_to(dinv2d.reshape(-1)[:, None], (NB, 128))
    fnode_col = _mlp2(nsf2d.reshape(NB, 1), fn_w1, fn_b1, fn_w2, fn_b2)
    fnode2d = fnode_col.reshape(80, 128)

    x_pad = jnp.concatenate(
        [x, jnp.zeros((NB - N, x.shape[1]), jnp.float32)], axis=0)
    y1 = _run_mm1(x_pad, W1, dinvb)
    a1 = _run_conv(y1, row1d, col1d)
    y2 = _run_comb_mm(a1, y1, dinvb, b1, W2)
    a2 = _run_conv(y2, row1d, col1d)
    y3 = _run_comb_mm(a2, y2, dinvb, b2, W3)
    a3 = _run_conv(y3, row1d, col1d)
    h3 = _run_comb3(a3, y3, dinvb, b3)

    src1d = edge[0].astype(jnp.int32)
    dst1d = edge[1].astype(jnp.int32)
    partials = _run_outfeat(h3, src1d, dst1d)

    records = _run_place(row1d, col1d, ew1d, fnode2d, bases_t)
    out0 = _run_isect(src1d, dst1d, base2d.reshape(-1), tot2d.reshape(-1),
                      records)

    s0_col, feat = _run_final(out0, partials)
    out_feat = feat.reshape(256)
    raw = _mlp2(s0_col, gp_w1, gp_b1, gp_w2, gp_b2)
    out_struct = jax.nn.sigmoid(raw)
    al = jax.nn.softmax(alpha)
    out = al[0] * out_struct + al[1] * out_feat + 1e-15
    return (out, out_struct, out_feat, raw)
